# baseline (speedup 1.0000x reference)
import functools

import jax
import jax.numpy as jnp
from jax import lax
from jax.experimental import pallas as pl
from jax.experimental.pallas import tpu as pltpu
from jax.experimental.pallas import tpu_sc as plsc

_BATCH = 4
_SEQ = 8192
_D = 1024
_NUM_CORES = 2
_NUM_SUBCORES = 16
_NW = _NUM_CORES * _NUM_SUBCORES
_ROWS_PER_W = _SEQ // _NW
_CHUNK = 32
_NCHUNK = _ROWS_PER_W // _CHUNK


@functools.partial(
    pl.kernel,
    mesh=plsc.VectorSubcoreMesh(core_axis_name="c", subcore_axis_name="s"),
    out_type=jax.ShapeDtypeStruct((_BATCH * _SEQ, _D), jnp.float32),
    scratch_types=[
        pltpu.VMEM((_CHUNK, _D), jnp.float32),
        pltpu.VMEM((_CHUNK, _D), jnp.float32),
        pltpu.SemaphoreType.DMA,
        pltpu.SemaphoreType.DMA,
    ],
)
def _sc_copy(emb_hbm, out_hbm, buf0, buf1, rd_sem, wr_sem):
    wid = lax.axis_index("s") * _NUM_CORES + lax.axis_index("c")
    base = wid * _ROWS_PER_W
    bufs = (buf0, buf1)

    def read(c):
        off = base + c * _CHUNK
        return pltpu.make_async_copy(
            emb_hbm.at[pl.ds(off, _CHUNK)], bufs[c % 2], rd_sem
        )

    def writes(c):
        off = base + c * _CHUNK
        return [
            pltpu.make_async_copy(
                bufs[c % 2], out_hbm.at[pl.ds(b * _SEQ + off, _CHUNK)], wr_sem
            )
            for b in range(_BATCH)
        ]

    read(0).start()
    pending = []
    for c in range(_NCHUNK):
        read(c).wait()
        ws = writes(c)
        for w in ws:
            w.start()
        if c + 1 < _NCHUNK:
            if pending:
                for w in pending:
                    w.wait()
            pending = ws
            read(c + 1).start()
    for w in pending:
        w.wait()
    for w in ws:
        w.wait()


def kernel(x, emb):
    batch, seq_len, d_model = x.shape
    out = _sc_copy(emb[:seq_len])
    return out.reshape(batch, seq_len, d_model)
